# initial kernel scaffold (unmeasured)
import jax
import jax.numpy as jnp
from jax import lax
from jax.experimental import pallas as pl
from jax.experimental.pallas import tpu as pltpu

N_DEV = 8
M, K, N = 8192, 1024, 4096
CH = M // N_DEV


def _matmul(x, w):
    BM, BN = 512, 1024

    def body(x_ref, w_ref, o_ref):
        o_ref[...] = jnp.dot(
            x_ref[...], w_ref[...], preferred_element_type=jnp.float32
        )

    return pl.pallas_call(
        body,
        grid=(M // BM, N // BN),
        in_specs=[
            pl.BlockSpec((BM, K), lambda i, j: (i, 0)),
            pl.BlockSpec((K, BN), lambda i, j: (0, j)),
        ],
        out_specs=pl.BlockSpec((BM, BN), lambda i, j: (i, j)),
        out_shape=jax.ShapeDtypeStruct((M, N), jnp.float32),
    )(x, w)


def _gelu(v):
    c = 0.7978845608028654
    return 0.5 * v * (1.0 + jnp.tanh(c * (v + 0.044715 * v * v * v)))


def _ar_body(p_ref, o_ref, comm_ref, sbuf_ref, va, vb,
             rs_send, rs_recv, ag_send, ag_recv, cp_sem):
    me = lax.axis_index("i")
    right = lax.rem(me + 1, N_DEV)
    left = lax.rem(me + N_DEV - 1, N_DEV)

    barrier = pltpu.get_barrier_semaphore()
    for nbr in (left, right):
        pl.semaphore_signal(
            barrier, inc=1, device_id=(nbr,),
            device_id_type=pl.DeviceIdType.MESH,
        )
    pl.semaphore_wait(barrier, 2)

    def chunk(ref, idx):
        return ref.at[pl.ds(idx * CH, CH), :]

    for s in range(N_DEV - 1):
        if s == 0:
            src = chunk(p_ref, me)
        else:
            idx = lax.rem(me - s + N_DEV, N_DEV)
            c1 = pltpu.make_async_copy(comm_ref.at[s - 1], va, cp_sem.at[0])
            c2 = pltpu.make_async_copy(chunk(p_ref, idx), vb, cp_sem.at[1])
            c1.start()
            c2.start()
            c1.wait()
            c2.wait()
            va[...] = va[...] + vb[...]
            c3 = pltpu.make_async_copy(va, sbuf_ref, cp_sem.at[0])
            c3.start()
            c3.wait()
            src = sbuf_ref
        rdma = pltpu.make_async_remote_copy(
            src_ref=src,
            dst_ref=comm_ref.at[s],
            send_sem=rs_send.at[s],
            recv_sem=rs_recv.at[s],
            device_id=(right,),
            device_id_type=pl.DeviceIdType.MESH,
        )
        rdma.start()
        rdma.wait()

    r = lax.rem(me + 1, N_DEV)
    c1 = pltpu.make_async_copy(comm_ref.at[N_DEV - 2], va, cp_sem.at[0])
    c2 = pltpu.make_async_copy(chunk(p_ref, r), vb, cp_sem.at[1])
    c1.start()
    c2.start()
    c1.wait()
    c2.wait()
    va[...] = _gelu(va[...] + vb[...])
    c3 = pltpu.make_async_copy(va, chunk(o_ref, r), cp_sem.at[0])
    c3.start()
    c3.wait()

    for s in range(N_DEV - 1):
        idx = lax.rem(me + 1 - s + N_DEV, N_DEV)
        rdma = pltpu.make_async_remote_copy(
            src_ref=chunk(o_ref, idx),
            dst_ref=chunk(o_ref, idx),
            send_sem=ag_send.at[s],
            recv_sem=ag_recv.at[s],
            device_id=(right,),
            device_id_type=pl.DeviceIdType.MESH,
        )
        rdma.start()
        rdma.wait()


def _allreduce_gelu(p):
    return pl.pallas_call(
        _ar_body,
        in_specs=[pl.BlockSpec(memory_space=pltpu.HBM)],
        out_specs=pl.BlockSpec(memory_space=pltpu.HBM),
        out_shape=jax.ShapeDtypeStruct((M, N), jnp.float32),
        scratch_shapes=[
            pltpu.HBM((N_DEV - 1, CH, N), jnp.float32),
            pltpu.HBM((CH, N), jnp.float32),
            pltpu.VMEM((CH, N), jnp.float32),
            pltpu.VMEM((CH, N), jnp.float32),
            pltpu.SemaphoreType.DMA((N_DEV - 1,)),
            pltpu.SemaphoreType.DMA((N_DEV - 1,)),
            pltpu.SemaphoreType.DMA((N_DEV - 1,)),
            pltpu.SemaphoreType.DMA((N_DEV - 1,)),
            pltpu.SemaphoreType.DMA((2,)),
        ],
        compiler_params=pltpu.CompilerParams(collective_id=0),
    )(p)


def kernel(x, w_mat):
    partial = _matmul(x, w_mat)
    return _allreduce_gelu(partial)


# baseline (device time: 2890771 ns/iter reference)
import jax
import jax.numpy as jnp
from jax import lax
from jax.experimental import pallas as pl
from jax.experimental.pallas import tpu as pltpu

N_DEV = 8
M, K, N = 8192, 1024, 4096
CH = M // N_DEV


def _matmul(x, w):
    BM, BN = 512, 1024

    def body(x_ref, w_ref, o_ref):
        o_ref[...] = jnp.dot(
            x_ref[...], w_ref[...], preferred_element_type=jnp.float32
        )

    return pl.pallas_call(
        body,
        grid=(M // BM, N // BN),
        in_specs=[
            pl.BlockSpec((BM, K), lambda i, j: (i, 0)),
            pl.BlockSpec((K, BN), lambda i, j: (0, j)),
        ],
        out_specs=pl.BlockSpec((BM, BN), lambda i, j: (i, j)),
        out_shape=jax.ShapeDtypeStruct((M, N), jnp.float32),
    )(x, w)


def _gelu(v):
    c = 0.7978845608028654
    return 0.5 * v * (1.0 + jnp.tanh(c * (v + 0.044715 * v * v * v)))


def _ar_body(p_ref, o_ref, comm_ref, sbuf_ref, va, vb,
             rs_send, rs_recv, ag_send, ag_recv, cp_sem):
    me = lax.axis_index("i")
    right = lax.rem(me + 1, N_DEV)
    left = lax.rem(me + N_DEV - 1, N_DEV)

    barrier = pltpu.get_barrier_semaphore()
    for nbr in (left, right):
        pl.semaphore_signal(
            barrier, inc=1, device_id=(nbr,),
            device_id_type=pl.DeviceIdType.MESH,
        )
    pl.semaphore_wait(barrier, 2)

    def chunk(ref, idx):
        return ref.at[pl.ds(idx * CH, CH), :]

    for s in range(N_DEV - 1):
        if s == 0:
            src = chunk(p_ref, me)
        else:
            idx = lax.rem(me - s + N_DEV, N_DEV)
            c1 = pltpu.make_async_copy(comm_ref.at[s - 1], va, cp_sem.at[0])
            c2 = pltpu.make_async_copy(chunk(p_ref, idx), vb, cp_sem.at[1])
            c1.start()
            c2.start()
            c1.wait()
            c2.wait()
            va[...] = va[...] + vb[...]
            c3 = pltpu.make_async_copy(va, sbuf_ref, cp_sem.at[0])
            c3.start()
            c3.wait()
            src = sbuf_ref
        rdma = pltpu.make_async_remote_copy(
            src_ref=src,
            dst_ref=comm_ref.at[s],
            send_sem=rs_send.at[s],
            recv_sem=rs_recv.at[s],
            device_id=(right,),
            device_id_type=pl.DeviceIdType.MESH,
        )
        rdma.start()
        rdma.wait()

    r = lax.rem(me + 1, N_DEV)
    c1 = pltpu.make_async_copy(comm_ref.at[N_DEV - 2], va, cp_sem.at[0])
    c2 = pltpu.make_async_copy(chunk(p_ref, r), vb, cp_sem.at[1])
    c1.start()
    c2.start()
    c1.wait()
    c2.wait()
    va[...] = _gelu(va[...] + vb[...])
    c3 = pltpu.make_async_copy(va, chunk(o_ref, r), cp_sem.at[0])
    c3.start()
    c3.wait()

    for s in range(N_DEV - 1):
        idx = lax.rem(me + 1 - s + N_DEV, N_DEV)
        rdma = pltpu.make_async_remote_copy(
            src_ref=chunk(o_ref, idx),
            dst_ref=chunk(o_ref, idx),
            send_sem=ag_send.at[s],
            recv_sem=ag_recv.at[s],
            device_id=(right,),
            device_id_type=pl.DeviceIdType.MESH,
        )
        rdma.start()
        rdma.wait()


def _allreduce_gelu(p):
    out, _, _ = pl.pallas_call(
        _ar_body,
        in_specs=[pl.BlockSpec(memory_space=pltpu.HBM)],
        out_specs=[
            pl.BlockSpec(memory_space=pltpu.HBM),
            pl.BlockSpec(memory_space=pltpu.HBM),
            pl.BlockSpec(memory_space=pltpu.HBM),
        ],
        out_shape=[
            jax.ShapeDtypeStruct((M, N), jnp.float32),
            jax.ShapeDtypeStruct((N_DEV - 1, CH, N), jnp.float32),
            jax.ShapeDtypeStruct((CH, N), jnp.float32),
        ],
        scratch_shapes=[
            pltpu.VMEM((CH, N), jnp.float32),
            pltpu.VMEM((CH, N), jnp.float32),
            pltpu.SemaphoreType.DMA((N_DEV - 1,)),
            pltpu.SemaphoreType.DMA((N_DEV - 1,)),
            pltpu.SemaphoreType.DMA((N_DEV - 1,)),
            pltpu.SemaphoreType.DMA((N_DEV - 1,)),
            pltpu.SemaphoreType.DMA((2,)),
        ],
        compiler_params=pltpu.CompilerParams(collective_id=0),
    )(p)
    return out


def kernel(x, w_mat):
    partial = _matmul(x, w_mat)
    return _allreduce_gelu(partial)


# device time: 1652762 ns/iter; 1.7491x vs baseline; 1.7491x over previous
import jax
import jax.numpy as jnp
from jax import lax
from jax.experimental import pallas as pl
from jax.experimental.pallas import tpu as pltpu

N_DEV = 8
M, K, N = 8192, 1024, 4096
CH = M // N_DEV


def _matmul(x, w):
    BM, BN = 512, 1024

    def body(x_ref, w_ref, o_ref):
        o_ref[...] = jnp.dot(
            x_ref[...], w_ref[...], preferred_element_type=jnp.float32
        )

    return pl.pallas_call(
        body,
        grid=(M // BM, N // BN),
        in_specs=[
            pl.BlockSpec((BM, K), lambda i, j: (i, 0)),
            pl.BlockSpec((K, BN), lambda i, j: (0, j)),
        ],
        out_specs=pl.BlockSpec((BM, BN), lambda i, j: (i, j)),
        out_shape=jax.ShapeDtypeStruct((M, N), jnp.float32),
    )(x, w)


def _gelu(v):
    c = 0.7978845608028654
    return 0.5 * v * (1.0 + jnp.tanh(c * (v + 0.044715 * v * v * v)))


HALF = N // 2


def _ar_body(p_ref, o_ref, comm_ref, sbuf_ref, va, vb,
             rs_send, rs_recv, ag_send, ag_recv, cp_sem):
    me = lax.axis_index("i")
    right = lax.rem(me + 1, N_DEV)
    left = lax.rem(me + N_DEV - 1, N_DEV)

    barrier = pltpu.get_barrier_semaphore()
    for nbr in (left, right):
        pl.semaphore_signal(
            barrier, inc=1, device_id=(nbr,),
            device_id_type=pl.DeviceIdType.MESH,
        )
    pl.semaphore_wait(barrier, 2)

    def blk(ref, idx, d):
        return ref.at[pl.ds(idx * CH, CH), pl.ds(d * HALF, HALF)]

    dirs = ((0, 1, right), (1, -1, left))

    for s in range(N_DEV - 1):
        rdmas = []
        for d, disp, tgt in dirs:
            if s == 0:
                src = blk(p_ref, me, d)
            else:
                idx = lax.rem(me - disp * s + 2 * N_DEV, N_DEV)
                c1 = pltpu.make_async_copy(
                    comm_ref.at[d, s - 1], va, cp_sem.at[0])
                c2 = pltpu.make_async_copy(
                    blk(p_ref, idx, d), vb, cp_sem.at[1])
                c1.start()
                c2.start()
                c1.wait()
                c2.wait()
                va[...] = va[...] + vb[...]
                c3 = pltpu.make_async_copy(va, sbuf_ref.at[d], cp_sem.at[0])
                c3.start()
                c3.wait()
                src = sbuf_ref.at[d]
            rdma = pltpu.make_async_remote_copy(
                src_ref=src,
                dst_ref=comm_ref.at[d, s],
                send_sem=rs_send.at[d, s],
                recv_sem=rs_recv.at[d, s],
                device_id=(tgt,),
                device_id_type=pl.DeviceIdType.MESH,
            )
            rdma.start()
            rdmas.append(rdma)
        for rdma in rdmas:
            rdma.wait()

    for d, disp, tgt in dirs:
        r = lax.rem(me + disp + N_DEV, N_DEV)
        c1 = pltpu.make_async_copy(comm_ref.at[d, N_DEV - 2], va, cp_sem.at[0])
        c2 = pltpu.make_async_copy(blk(p_ref, r, d), vb, cp_sem.at[1])
        c1.start()
        c2.start()
        c1.wait()
        c2.wait()
        va[...] = _gelu(va[...] + vb[...])
        c3 = pltpu.make_async_copy(va, blk(o_ref, r, d), cp_sem.at[0])
        c3.start()
        c3.wait()

    for s in range(N_DEV - 1):
        rdmas = []
        for d, disp, tgt in dirs:
            idx = lax.rem(me + disp - disp * s + 2 * N_DEV, N_DEV)
            rdma = pltpu.make_async_remote_copy(
                src_ref=blk(o_ref, idx, d),
                dst_ref=blk(o_ref, idx, d),
                send_sem=ag_send.at[d, s],
                recv_sem=ag_recv.at[d, s],
                device_id=(tgt,),
                device_id_type=pl.DeviceIdType.MESH,
            )
            rdma.start()
            rdmas.append(rdma)
        for rdma in rdmas:
            rdma.wait()


def _allreduce_gelu(p):
    out, _, _ = pl.pallas_call(
        _ar_body,
        in_specs=[pl.BlockSpec(memory_space=pltpu.HBM)],
        out_specs=[
            pl.BlockSpec(memory_space=pltpu.HBM),
            pl.BlockSpec(memory_space=pltpu.HBM),
            pl.BlockSpec(memory_space=pltpu.HBM),
        ],
        out_shape=[
            jax.ShapeDtypeStruct((M, N), jnp.float32),
            jax.ShapeDtypeStruct((2, N_DEV - 1, CH, HALF), jnp.float32),
            jax.ShapeDtypeStruct((2, CH, HALF), jnp.float32),
        ],
        scratch_shapes=[
            pltpu.VMEM((CH, HALF), jnp.float32),
            pltpu.VMEM((CH, HALF), jnp.float32),
            pltpu.SemaphoreType.DMA((2, N_DEV - 1)),
            pltpu.SemaphoreType.DMA((2, N_DEV - 1)),
            pltpu.SemaphoreType.DMA((2, N_DEV - 1)),
            pltpu.SemaphoreType.DMA((2, N_DEV - 1)),
            pltpu.SemaphoreType.DMA((2,)),
        ],
        compiler_params=pltpu.CompilerParams(collective_id=0),
    )(p)
    return out


def kernel(x, w_mat):
    partial = _matmul(x, w_mat)
    return _allreduce_gelu(partial)


# device time: 1610469 ns/iter; 1.7950x vs baseline; 1.0263x over previous
import jax
import jax.numpy as jnp
from jax import lax
from jax.experimental import pallas as pl
from jax.experimental.pallas import tpu as pltpu

N_DEV = 8
M, K, N = 8192, 1024, 4096
CH = M // N_DEV


def _matmul(x, w):
    BM, BN = 512, 1024

    def body(x_ref, w_ref, o_ref):
        o_ref[...] = jnp.dot(
            x_ref[...], w_ref[...], preferred_element_type=jnp.float32
        )

    return pl.pallas_call(
        body,
        grid=(M // BM, N // BN),
        in_specs=[
            pl.BlockSpec((BM, K), lambda i, j: (i, 0)),
            pl.BlockSpec((K, BN), lambda i, j: (0, j)),
        ],
        out_specs=pl.BlockSpec((BM, BN), lambda i, j: (i, j)),
        out_shape=jax.ShapeDtypeStruct((M, N), jnp.float32),
    )(x, w)


def _gelu(v):
    c = 0.7978845608028654
    return 0.5 * v * (1.0 + jnp.tanh(c * (v + 0.044715 * v * v * v)))


HALF = N // 2


def _ar_body(p_ref, o_ref, comm_ref, sv, va, vb,
             rs_send, rs_recv, ag_send, ag_recv, cp_sem):
    me = lax.axis_index("i")
    right = lax.rem(me + 1, N_DEV)
    left = lax.rem(me + N_DEV - 1, N_DEV)

    barrier = pltpu.get_barrier_semaphore()
    for nbr in (left, right):
        pl.semaphore_signal(
            barrier, inc=1, device_id=(nbr,),
            device_id_type=pl.DeviceIdType.MESH,
        )
    pl.semaphore_wait(barrier, 2)

    def blk(ref, idx, d):
        return ref.at[pl.ds(idx * CH, CH), pl.ds(d * HALF, HALF)]

    dirs = ((0, 1, right), (1, -1, left))

    for s in range(N_DEV - 1):
        rdmas = []
        for d, disp, tgt in dirs:
            if s == 0:
                src = blk(p_ref, me, d)
            else:
                idx = lax.rem(me - disp * s + 2 * N_DEV, N_DEV)
                c1 = pltpu.make_async_copy(
                    comm_ref.at[d, s - 1], va, cp_sem.at[0])
                c2 = pltpu.make_async_copy(
                    blk(p_ref, idx, d), vb, cp_sem.at[1])
                c1.start()
                c2.start()
                c1.wait()
                c2.wait()
                sv[d, :, :] = va[...] + vb[...]
                src = sv.at[d]
            rdma = pltpu.make_async_remote_copy(
                src_ref=src,
                dst_ref=comm_ref.at[d, s],
                send_sem=rs_send.at[d, s],
                recv_sem=rs_recv.at[d, s],
                device_id=(tgt,),
                device_id_type=pl.DeviceIdType.MESH,
            )
            rdma.start()
            rdmas.append(rdma)
        for rdma in rdmas:
            rdma.wait()

    for d, disp, tgt in dirs:
        r = lax.rem(me + disp + N_DEV, N_DEV)
        c1 = pltpu.make_async_copy(comm_ref.at[d, N_DEV - 2], va, cp_sem.at[0])
        c2 = pltpu.make_async_copy(blk(p_ref, r, d), vb, cp_sem.at[1])
        c1.start()
        c2.start()
        c1.wait()
        c2.wait()
        va[...] = _gelu(va[...] + vb[...])
        c3 = pltpu.make_async_copy(va, blk(o_ref, r, d), cp_sem.at[0])
        c3.start()
        c3.wait()

    for s in range(N_DEV - 1):
        rdmas = []
        for d, disp, tgt in dirs:
            idx = lax.rem(me + disp - disp * s + 2 * N_DEV, N_DEV)
            rdma = pltpu.make_async_remote_copy(
                src_ref=blk(o_ref, idx, d),
                dst_ref=blk(o_ref, idx, d),
                send_sem=ag_send.at[d, s],
                recv_sem=ag_recv.at[d, s],
                device_id=(tgt,),
                device_id_type=pl.DeviceIdType.MESH,
            )
            rdma.start()
            rdmas.append(rdma)
        for rdma in rdmas:
            rdma.wait()


def _allreduce_gelu(p):
    out, _ = pl.pallas_call(
        _ar_body,
        in_specs=[pl.BlockSpec(memory_space=pltpu.HBM)],
        out_specs=[
            pl.BlockSpec(memory_space=pltpu.HBM),
            pl.BlockSpec(memory_space=pltpu.HBM),
        ],
        out_shape=[
            jax.ShapeDtypeStruct((M, N), jnp.float32),
            jax.ShapeDtypeStruct((2, N_DEV - 1, CH, HALF), jnp.float32),
        ],
        scratch_shapes=[
            pltpu.VMEM((2, CH, HALF), jnp.float32),
            pltpu.VMEM((CH, HALF), jnp.float32),
            pltpu.VMEM((CH, HALF), jnp.float32),
            pltpu.SemaphoreType.DMA((2, N_DEV - 1)),
            pltpu.SemaphoreType.DMA((2, N_DEV - 1)),
            pltpu.SemaphoreType.DMA((2, N_DEV - 1)),
            pltpu.SemaphoreType.DMA((2, N_DEV - 1)),
            pltpu.SemaphoreType.DMA((2,)),
        ],
        compiler_params=pltpu.CompilerParams(collective_id=0),
    )(p)
    return out


def kernel(x, w_mat):
    partial = _matmul(x, w_mat)
    return _allreduce_gelu(partial)


# device time: 1503193 ns/iter; 1.9231x vs baseline; 1.0714x over previous
import jax
import jax.numpy as jnp
from jax import lax
from jax.experimental import pallas as pl
from jax.experimental.pallas import tpu as pltpu

N_DEV = 8
M, K, N = 8192, 1024, 4096
CH = M // N_DEV
HALF = N // 2


def _gelu(v):
    c = 0.7978845608028654
    return 0.5 * v * (1.0 + jnp.tanh(c * (v + 0.044715 * v * v * v)))


def _body(x_ref, w_ref, o_ref, p_ref, comm_ref,
          sv, vx, vr, vp,
          rs_send, rs_recv, ag_send, ag_recv, cp_sem):
    me = lax.axis_index("i")
    right = lax.rem(me + 1, N_DEV)
    left = lax.rem(me + N_DEV - 1, N_DEV)

    def blk(ref, idx, d):
        return ref.at[pl.ds(idx * CH, CH), pl.ds(d * HALF, HALF)]

    def xchunk(idx):
        return x_ref.at[pl.ds(idx * CH, CH), :]

    c0 = pltpu.make_async_copy(xchunk(me), vx.at[0], cp_sem.at[0])
    c0.start()
    c0.wait()
    for d in (0, 1):
        sv[d, :, :] = jnp.dot(
            vx[0], w_ref[:, d * HALF:(d + 1) * HALF],
            preferred_element_type=jnp.float32,
        )

    barrier = pltpu.get_barrier_semaphore()
    for nbr in (left, right):
        pl.semaphore_signal(
            barrier, inc=1, device_id=(nbr,),
            device_id_type=pl.DeviceIdType.MESH,
        )
    pl.semaphore_wait(barrier, 2)

    dirs = ((0, 1, right), (1, -1, left))

    inflight = []
    for d, disp, tgt in dirs:
        rdma = pltpu.make_async_remote_copy(
            src_ref=sv.at[d],
            dst_ref=comm_ref.at[d, 0],
            send_sem=rs_send.at[d, 0],
            recv_sem=rs_recv.at[d, 0],
            device_id=(tgt,),
            device_id_type=pl.DeviceIdType.MESH,
        )
        rdma.start()
        inflight.append(rdma)

    offsets = (1, -1, 2, -2, 3, -3, 4)
    pf = [None, None]
    idx0 = lax.rem(me + offsets[0] + N_DEV, N_DEV)
    pf[0] = pltpu.make_async_copy(xchunk(idx0), vx.at[0], cp_sem.at[0])
    pf[0].start()
    outw = [None, None]
    for n, off in enumerate(offsets):
        idx = lax.rem(me + off + N_DEV, N_DEV)
        cur = n % 2
        pf[cur].wait()
        if n + 1 < len(offsets):
            nidx = lax.rem(me + offsets[n + 1] + N_DEV, N_DEV)
            nxt = (n + 1) % 2
            pf[nxt] = pltpu.make_async_copy(
                xchunk(nidx), vx.at[nxt], cp_sem.at[nxt])
            pf[nxt].start()
        for d, buf in ((0, vp), (1, vr)):
            if outw[d] is not None:
                outw[d].wait()
            buf[...] = jnp.dot(
                vx[cur], w_ref[:, d * HALF:(d + 1) * HALF],
                preferred_element_type=jnp.float32,
            )
            outw[d] = pltpu.make_async_copy(
                buf, blk(p_ref, idx, d), cp_sem.at[2 + d])
            outw[d].start()
    for d in (0, 1):
        outw[d].wait()

    for s in range(1, N_DEV - 1):
        for rdma in inflight:
            rdma.wait()
        inflight = []
        for d, disp, tgt in dirs:
            idx = lax.rem(me - disp * s + 2 * N_DEV, N_DEV)
            c1 = pltpu.make_async_copy(comm_ref.at[d, s - 1], vr, cp_sem.at[0])
            c2 = pltpu.make_async_copy(blk(p_ref, idx, d), vp, cp_sem.at[1])
            c1.start()
            c2.start()
            c1.wait()
            c2.wait()
            sv[d, :, :] = vr[...] + vp[...]
            rdma = pltpu.make_async_remote_copy(
                src_ref=sv.at[d],
                dst_ref=comm_ref.at[d, s],
                send_sem=rs_send.at[d, s],
                recv_sem=rs_recv.at[d, s],
                device_id=(tgt,),
                device_id_type=pl.DeviceIdType.MESH,
            )
            rdma.start()
            inflight.append(rdma)
    for rdma in inflight:
        rdma.wait()

    for d, disp, tgt in dirs:
        r = lax.rem(me + disp + N_DEV, N_DEV)
        c1 = pltpu.make_async_copy(comm_ref.at[d, N_DEV - 2], vr, cp_sem.at[0])
        c2 = pltpu.make_async_copy(blk(p_ref, r, d), vp, cp_sem.at[1])
        c1.start()
        c2.start()
        c1.wait()
        c2.wait()
        vp[...] = _gelu(vr[...] + vp[...])
        c3 = pltpu.make_async_copy(vp, blk(o_ref, r, d), cp_sem.at[0])
        c3.start()
        c3.wait()

    for s in range(N_DEV - 1):
        rdmas = []
        for d, disp, tgt in dirs:
            idx = lax.rem(me + disp - disp * s + 2 * N_DEV, N_DEV)
            rdma = pltpu.make_async_remote_copy(
                src_ref=blk(o_ref, idx, d),
                dst_ref=blk(o_ref, idx, d),
                send_sem=ag_send.at[d, s],
                recv_sem=ag_recv.at[d, s],
                device_id=(tgt,),
                device_id_type=pl.DeviceIdType.MESH,
            )
            rdma.start()
            rdmas.append(rdma)
        for rdma in rdmas:
            rdma.wait()


def kernel(x, w_mat):
    out, _, _ = pl.pallas_call(
        _body,
        in_specs=[
            pl.BlockSpec(memory_space=pltpu.HBM),
            pl.BlockSpec(memory_space=pltpu.VMEM),
        ],
        out_specs=[
            pl.BlockSpec(memory_space=pltpu.HBM),
            pl.BlockSpec(memory_space=pltpu.HBM),
            pl.BlockSpec(memory_space=pltpu.HBM),
        ],
        out_shape=[
            jax.ShapeDtypeStruct((M, N), jnp.float32),
            jax.ShapeDtypeStruct((M, N), jnp.float32),
            jax.ShapeDtypeStruct((2, N_DEV - 1, CH, HALF), jnp.float32),
        ],
        scratch_shapes=[
            pltpu.VMEM((2, CH, HALF), jnp.float32),
            pltpu.VMEM((2, CH, K), jnp.float32),
            pltpu.VMEM((CH, HALF), jnp.float32),
            pltpu.VMEM((CH, HALF), jnp.float32),
            pltpu.SemaphoreType.DMA((2, N_DEV - 1)),
            pltpu.SemaphoreType.DMA((2, N_DEV - 1)),
            pltpu.SemaphoreType.DMA((2, N_DEV - 1)),
            pltpu.SemaphoreType.DMA((2, N_DEV - 1)),
            pltpu.SemaphoreType.DMA((4,)),
        ],
        compiler_params=pltpu.CompilerParams(
            collective_id=0, vmem_limit_bytes=100 * 1024 * 1024
        ),
    )(x, w_mat)
    return out


# device time: 1481114 ns/iter; 1.9518x vs baseline; 1.0149x over previous
import jax
import jax.numpy as jnp
from jax import lax
from jax.experimental import pallas as pl
from jax.experimental.pallas import tpu as pltpu

N_DEV = 8
M, K, N = 8192, 1024, 4096
CH = M // N_DEV
HALF = N // 2
TR = 512


def _gelu(v):
    c = 0.7978845608028654
    return 0.5 * v * (1.0 + jnp.tanh(c * (v + 0.044715 * v * v * v)))


def _body(x_ref, w_ref, o_ref, p_ref, comm_ref,
          sv, vp, vx, vm,
          rs_send, rs_recv, ag_send, ag_recv, cp_sem):
    me = lax.axis_index("i")
    right = lax.rem(me + 1, N_DEV)
    left = lax.rem(me + N_DEV - 1, N_DEV)

    def blk(ref, idx, d):
        return ref.at[pl.ds(idx * CH, CH), pl.ds(d * HALF, HALF)]

    def xchunk(idx):
        return x_ref.at[pl.ds(idx * CH, CH), :]

    def wcols(d):
        return w_ref[:, d * HALF:(d + 1) * HALF]

    c0 = pltpu.make_async_copy(xchunk(me), vx.at[0], cp_sem.at[4])
    c0.start()
    c0.wait()
    for d in (0, 1):
        sv[d, :, :] = jnp.dot(
            vx[0], wcols(d), preferred_element_type=jnp.float32)

    barrier = pltpu.get_barrier_semaphore()
    for nbr in (left, right):
        pl.semaphore_signal(
            barrier, inc=1, device_id=(nbr,),
            device_id_type=pl.DeviceIdType.MESH,
        )
    pl.semaphore_wait(barrier, 2)

    dirs = ((0, 1, right), (1, -1, left))

    def start_rs(s):
        out = []
        for d, disp, tgt in dirs:
            rdma = pltpu.make_async_remote_copy(
                src_ref=sv.at[d],
                dst_ref=comm_ref.at[d, s],
                send_sem=rs_send.at[d, s],
                recv_sem=rs_recv.at[d, s],
                device_id=(tgt,),
                device_id_type=pl.DeviceIdType.MESH,
            )
            rdma.start()
            out.append(rdma)
        return out

    def mm_partials(s):
        xc = []
        for d, disp, tgt in dirs:
            idx = lax.rem(me - disp * (s + 1) + 2 * N_DEV, N_DEV)
            c = pltpu.make_async_copy(xchunk(idx), vx.at[d], cp_sem.at[4 + d])
            c.start()
            xc.append((c, idx))
        for d, disp, tgt in dirs:
            c, idx = xc[d]
            c.wait()
            for t in range(CH // TR):
                vm[...] = jnp.dot(
                    vx[d, t * TR:(t + 1) * TR, :], wcols(d),
                    preferred_element_type=jnp.float32)
                co = pltpu.make_async_copy(
                    vm,
                    p_ref.at[pl.ds(idx * CH + t * TR, TR),
                             pl.ds(d * HALF, HALF)],
                    cp_sem.at[6],
                )
                co.start()
                co.wait()

    inflight = start_rs(0)
    mm_partials(0)
    for s in range(1, N_DEV - 1):
        for rdma in inflight:
            rdma.wait()
        cps = []
        for d, disp, tgt in dirs:
            idx = lax.rem(me - disp * s + 2 * N_DEV, N_DEV)
            c1 = pltpu.make_async_copy(
                comm_ref.at[d, s - 1], sv.at[d], cp_sem.at[d])
            c2 = pltpu.make_async_copy(
                blk(p_ref, idx, d), vp.at[d], cp_sem.at[2 + d])
            c1.start()
            c2.start()
            cps.append((c1, c2))
        inflight = []
        for d, disp, tgt in dirs:
            c1, c2 = cps[d]
            c1.wait()
            c2.wait()
            sv[d, :, :] = sv[d] + vp[d]
            rdma = pltpu.make_async_remote_copy(
                src_ref=sv.at[d],
                dst_ref=comm_ref.at[d, s],
                send_sem=rs_send.at[d, s],
                recv_sem=rs_recv.at[d, s],
                device_id=(tgt,),
                device_id_type=pl.DeviceIdType.MESH,
            )
            rdma.start()
            inflight.append(rdma)
        mm_partials(s)
    for rdma in inflight:
        rdma.wait()

    cps = []
    for d, disp, tgt in dirs:
        r = lax.rem(me + disp + N_DEV, N_DEV)
        c1 = pltpu.make_async_copy(
            comm_ref.at[d, N_DEV - 2], sv.at[d], cp_sem.at[d])
        c2 = pltpu.make_async_copy(blk(p_ref, r, d), vp.at[d], cp_sem.at[2 + d])
        c1.start()
        c2.start()
        cps.append((c1, c2, r))
    inflight = []
    stores = []
    for d, disp, tgt in dirs:
        c1, c2, r = cps[d]
        c1.wait()
        c2.wait()
        sv[d, :, :] = _gelu(sv[d] + vp[d])
        rdma = pltpu.make_async_remote_copy(
            src_ref=sv.at[d],
            dst_ref=blk(o_ref, r, d),
            send_sem=ag_send.at[d, 0],
            recv_sem=ag_recv.at[d, 0],
            device_id=(tgt,),
            device_id_type=pl.DeviceIdType.MESH,
        )
        rdma.start()
        inflight.append(rdma)
        st = pltpu.make_async_copy(sv.at[d], blk(o_ref, r, d), cp_sem.at[4 + d])
        st.start()
        stores.append(st)

    for s in range(1, N_DEV - 1):
        for rdma in inflight:
            rdma.wait()
        inflight = []
        for d, disp, tgt in dirs:
            idx = lax.rem(me + disp - disp * s + 2 * N_DEV, N_DEV)
            rdma = pltpu.make_async_remote_copy(
                src_ref=blk(o_ref, idx, d),
                dst_ref=blk(o_ref, idx, d),
                send_sem=ag_send.at[d, s],
                recv_sem=ag_recv.at[d, s],
                device_id=(tgt,),
                device_id_type=pl.DeviceIdType.MESH,
            )
            rdma.start()
            inflight.append(rdma)
    for rdma in inflight:
        rdma.wait()
    for st in stores:
        st.wait()


def kernel(x, w_mat):
    out, _, _ = pl.pallas_call(
        _body,
        in_specs=[
            pl.BlockSpec(memory_space=pltpu.HBM),
            pl.BlockSpec(memory_space=pltpu.VMEM),
        ],
        out_specs=[
            pl.BlockSpec(memory_space=pltpu.HBM),
            pl.BlockSpec(memory_space=pltpu.HBM),
            pl.BlockSpec(memory_space=pltpu.HBM),
        ],
        out_shape=[
            jax.ShapeDtypeStruct((M, N), jnp.float32),
            jax.ShapeDtypeStruct((M, N), jnp.float32),
            jax.ShapeDtypeStruct((2, N_DEV - 1, CH, HALF), jnp.float32),
        ],
        scratch_shapes=[
            pltpu.VMEM((2, CH, HALF), jnp.float32),
            pltpu.VMEM((2, CH, HALF), jnp.float32),
            pltpu.VMEM((2, CH, K), jnp.float32),
            pltpu.VMEM((TR, HALF), jnp.float32),
            pltpu.SemaphoreType.DMA((2, N_DEV - 1)),
            pltpu.SemaphoreType.DMA((2, N_DEV - 1)),
            pltpu.SemaphoreType.DMA((2, N_DEV - 1)),
            pltpu.SemaphoreType.DMA((2, N_DEV - 1)),
            pltpu.SemaphoreType.DMA((8,)),
        ],
        compiler_params=pltpu.CompilerParams(
            collective_id=0, vmem_limit_bytes=100 * 1024 * 1024
        ),
    )(x, w_mat)
    return out


# device time: 1447370 ns/iter; 1.9973x vs baseline; 1.0233x over previous
import jax
import jax.numpy as jnp
from jax import lax
from jax.experimental import pallas as pl
from jax.experimental.pallas import tpu as pltpu

N_DEV = 8
M, K, N = 8192, 1024, 4096
CH = M // N_DEV
HALF = N // 2
TR = 512


def _gelu(v):
    c = 0.7978845608028654
    return 0.5 * v * (1.0 + jnp.tanh(c * (v + 0.044715 * v * v * v)))


def _body(x_ref, w_ref, o_ref, p_ref, comm_ref,
          sv, vp, vx, vm,
          rs_send, rs_recv, ag_send, ag_recv, cp_sem):
    me = lax.axis_index("i")
    right = lax.rem(me + 1, N_DEV)
    left = lax.rem(me + N_DEV - 1, N_DEV)

    def blk(ref, idx, d):
        return ref.at[pl.ds(idx * CH, CH), pl.ds(d * HALF, HALF)]

    def xchunk(idx):
        return x_ref.at[pl.ds(idx * CH, CH), :]

    def wcols(d):
        return w_ref[:, d * HALF:(d + 1) * HALF]

    c0 = pltpu.make_async_copy(xchunk(me), vx.at[0], cp_sem.at[4])
    c0.start()
    c0.wait()
    for d in (0, 1):
        sv[d, :, :] = jnp.dot(
            vx[0], wcols(d), preferred_element_type=jnp.float32)

    barrier = pltpu.get_barrier_semaphore()
    for nbr in (left, right):
        pl.semaphore_signal(
            barrier, inc=1, device_id=(nbr,),
            device_id_type=pl.DeviceIdType.MESH,
        )
    pl.semaphore_wait(barrier, 2)

    dirs = ((0, 1, right), (1, -1, left))

    def start_rs(s):
        out = []
        for d, disp, tgt in dirs:
            rdma = pltpu.make_async_remote_copy(
                src_ref=sv.at[d],
                dst_ref=comm_ref.at[d, s],
                send_sem=rs_send.at[d, s],
                recv_sem=rs_recv.at[d, s],
                device_id=(tgt,),
                device_id_type=pl.DeviceIdType.MESH,
            )
            rdma.start()
            out.append(rdma)
        return out

    def mm_partials(s):
        xc = []
        for d, disp, tgt in dirs:
            idx = lax.rem(me - disp * (s + 1) + 2 * N_DEV, N_DEV)
            c = pltpu.make_async_copy(xchunk(idx), vx.at[d], cp_sem.at[4 + d])
            c.start()
            xc.append((c, idx))
        for d, disp, tgt in dirs:
            c, idx = xc[d]
            c.wait()
            for t in range(CH // TR):
                vm[...] = jnp.dot(
                    vx[d, t * TR:(t + 1) * TR, :], wcols(d),
                    preferred_element_type=jnp.float32)
                co = pltpu.make_async_copy(
                    vm,
                    p_ref.at[pl.ds(idx * CH + t * TR, TR),
                             pl.ds(d * HALF, HALF)],
                    cp_sem.at[6],
                )
                co.start()
                co.wait()

    def prefetch_vp(s):
        cs = []
        for d, disp, tgt in dirs:
            idx = lax.rem(me - disp * s + 2 * N_DEV, N_DEV)
            c = pltpu.make_async_copy(
                blk(p_ref, idx, d), vp.at[d], cp_sem.at[2 + d])
            c.start()
            cs.append(c)
        return cs

    inflight = start_rs(0)
    mm_partials(0)
    vpc = prefetch_vp(1)
    for s in range(1, N_DEV - 1):
        for rdma in inflight:
            rdma.wait()
        c1s = []
        for d, disp, tgt in dirs:
            c1 = pltpu.make_async_copy(
                comm_ref.at[d, s - 1], sv.at[d], cp_sem.at[d])
            c1.start()
            c1s.append(c1)
        inflight = []
        for d, disp, tgt in dirs:
            c1s[d].wait()
            vpc[d].wait()
            sv[d, :, :] = sv[d] + vp[d]
            rdma = pltpu.make_async_remote_copy(
                src_ref=sv.at[d],
                dst_ref=comm_ref.at[d, s],
                send_sem=rs_send.at[d, s],
                recv_sem=rs_recv.at[d, s],
                device_id=(tgt,),
                device_id_type=pl.DeviceIdType.MESH,
            )
            rdma.start()
            inflight.append(rdma)
        mm_partials(s)
        vpc = prefetch_vp(s + 1)
    for rdma in inflight:
        rdma.wait()

    c1s = []
    for d, disp, tgt in dirs:
        c1 = pltpu.make_async_copy(
            comm_ref.at[d, N_DEV - 2], sv.at[d], cp_sem.at[d])
        c1.start()
        c1s.append(c1)
    inflight = []
    stores = []
    for d, disp, tgt in dirs:
        r = lax.rem(me + disp + N_DEV, N_DEV)
        c1s[d].wait()
        vpc[d].wait()
        sv[d, :, :] = _gelu(sv[d] + vp[d])
        rdma = pltpu.make_async_remote_copy(
            src_ref=sv.at[d],
            dst_ref=blk(o_ref, r, d),
            send_sem=ag_send.at[d, 0],
            recv_sem=ag_recv.at[d, 0],
            device_id=(tgt,),
            device_id_type=pl.DeviceIdType.MESH,
        )
        rdma.start()
        inflight.append(rdma)
        st = pltpu.make_async_copy(sv.at[d], blk(o_ref, r, d), cp_sem.at[4 + d])
        st.start()
        stores.append(st)

    for s in range(1, N_DEV - 1):
        for rdma in inflight:
            rdma.wait()
        inflight = []
        for d, disp, tgt in dirs:
            idx = lax.rem(me + disp - disp * s + 2 * N_DEV, N_DEV)
            rdma = pltpu.make_async_remote_copy(
                src_ref=blk(o_ref, idx, d),
                dst_ref=blk(o_ref, idx, d),
                send_sem=ag_send.at[d, s],
                recv_sem=ag_recv.at[d, s],
                device_id=(tgt,),
                device_id_type=pl.DeviceIdType.MESH,
            )
            rdma.start()
            inflight.append(rdma)
    for rdma in inflight:
        rdma.wait()
    for st in stores:
        st.wait()


def kernel(x, w_mat):
    out, _, _ = pl.pallas_call(
        _body,
        in_specs=[
            pl.BlockSpec(memory_space=pltpu.HBM),
            pl.BlockSpec(memory_space=pltpu.VMEM),
        ],
        out_specs=[
            pl.BlockSpec(memory_space=pltpu.HBM),
            pl.BlockSpec(memory_space=pltpu.HBM),
            pl.BlockSpec(memory_space=pltpu.HBM),
        ],
        out_shape=[
            jax.ShapeDtypeStruct((M, N), jnp.float32),
            jax.ShapeDtypeStruct((M, N), jnp.float32),
            jax.ShapeDtypeStruct((2, N_DEV - 1, CH, HALF), jnp.float32),
        ],
        scratch_shapes=[
            pltpu.VMEM((2, CH, HALF), jnp.float32),
            pltpu.VMEM((2, CH, HALF), jnp.float32),
            pltpu.VMEM((2, CH, K), jnp.float32),
            pltpu.VMEM((TR, HALF), jnp.float32),
            pltpu.SemaphoreType.DMA((2, N_DEV - 1)),
            pltpu.SemaphoreType.DMA((2, N_DEV - 1)),
            pltpu.SemaphoreType.DMA((2, N_DEV - 1)),
            pltpu.SemaphoreType.DMA((2, N_DEV - 1)),
            pltpu.SemaphoreType.DMA((8,)),
        ],
        compiler_params=pltpu.CompilerParams(
            collective_id=0, vmem_limit_bytes=100 * 1024 * 1024
        ),
    )(x, w_mat)
    return out


# device time: 1445740 ns/iter; 1.9995x vs baseline; 1.0011x over previous
import jax
import jax.numpy as jnp
from jax import lax
from jax.experimental import pallas as pl
from jax.experimental.pallas import tpu as pltpu

N_DEV = 8
M, K, N = 8192, 1024, 4096
CH = M // N_DEV
HALF = N // 2


def _gelu(v):
    c = 0.7978845608028654
    return 0.5 * v * (1.0 + jnp.tanh(c * (v + 0.044715 * v * v * v)))


def _body(x_ref, w_ref, o_ref, comm_ref,
          sv, vp, vx,
          rs_send, rs_recv, ag_send, ag_recv, cp_sem):
    me = lax.axis_index("i")
    right = lax.rem(me + 1, N_DEV)
    left = lax.rem(me + N_DEV - 1, N_DEV)

    barrier = pltpu.get_barrier_semaphore()
    for nbr in (left, right):
        pl.semaphore_signal(
            barrier, inc=1, device_id=(nbr,),
            device_id_type=pl.DeviceIdType.MESH,
        )

    def blk(ref, idx, d):
        return ref.at[pl.ds(idx * CH, CH), pl.ds(d * HALF, HALF)]

    def xchunk(idx):
        return x_ref.at[pl.ds(idx * CH, CH), :]

    def wcols(d):
        return w_ref[:, d * HALF:(d + 1) * HALF]

    dirs = ((0, 1, right), (1, -1, left))

    def rs_rdma(d, tgt, s):
        return pltpu.make_async_remote_copy(
            src_ref=sv.at[d],
            dst_ref=comm_ref.at[d, s],
            send_sem=rs_send.at[d, s],
            recv_sem=rs_recv.at[d, s],
            device_id=(tgt,),
            device_id_type=pl.DeviceIdType.MESH,
        )

    c0 = pltpu.make_async_copy(xchunk(me), vx.at[0], cp_sem.at[4])
    c0.start()
    c0.wait()
    inflight = []
    for d, disp, tgt in dirs:
        sv[d, :, :] = jnp.dot(
            vx[0], wcols(d), preferred_element_type=jnp.float32)
        if d == 0:
            pl.semaphore_wait(barrier, 2)
        rdma = rs_rdma(d, tgt, 0)
        rdma.start()
        inflight.append(rdma)

    def mm_partials(s):
        xc = []
        for d, disp, tgt in dirs:
            idx = lax.rem(me - disp * (s + 1) + 2 * N_DEV, N_DEV)
            c = pltpu.make_async_copy(xchunk(idx), vx.at[d], cp_sem.at[4 + d])
            c.start()
            xc.append(c)
        for d, disp, tgt in dirs:
            xc[d].wait()
            vp[d, :, :] = jnp.dot(
                vx[d], wcols(d), preferred_element_type=jnp.float32)

    mm_partials(0)
    for s in range(1, N_DEV - 1):
        for rdma in inflight:
            rdma.wait()
        c1s = []
        for d, disp, tgt in dirs:
            c1 = pltpu.make_async_copy(
                comm_ref.at[d, s - 1], sv.at[d], cp_sem.at[d])
            c1.start()
            c1s.append(c1)
        inflight = []
        for d, disp, tgt in dirs:
            c1s[d].wait()
            sv[d, :, :] = sv[d] + vp[d]
            rdma = rs_rdma(d, tgt, s)
            rdma.start()
            inflight.append(rdma)
        mm_partials(s)
    for rdma in inflight:
        rdma.wait()

    c1s = []
    for d, disp, tgt in dirs:
        c1 = pltpu.make_async_copy(
            comm_ref.at[d, N_DEV - 2], sv.at[d], cp_sem.at[d])
        c1.start()
        c1s.append(c1)
    inflight = []
    stores = []
    for d, disp, tgt in dirs:
        r = lax.rem(me + disp + N_DEV, N_DEV)
        c1s[d].wait()
        sv[d, :, :] = _gelu(sv[d] + vp[d])
        rdma = pltpu.make_async_remote_copy(
            src_ref=sv.at[d],
            dst_ref=blk(o_ref, r, d),
            send_sem=ag_send.at[d, 0],
            recv_sem=ag_recv.at[d, 0],
            device_id=(tgt,),
            device_id_type=pl.DeviceIdType.MESH,
        )
        rdma.start()
        inflight.append(rdma)
        st = pltpu.make_async_copy(sv.at[d], blk(o_ref, r, d), cp_sem.at[4 + d])
        st.start()
        stores.append(st)

    for s in range(1, N_DEV - 1):
        for rdma in inflight:
            rdma.wait()
        inflight = []
        for d, disp, tgt in dirs:
            idx = lax.rem(me + disp - disp * s + 2 * N_DEV, N_DEV)
            rdma = pltpu.make_async_remote_copy(
                src_ref=blk(o_ref, idx, d),
                dst_ref=blk(o_ref, idx, d),
                send_sem=ag_send.at[d, s],
                recv_sem=ag_recv.at[d, s],
                device_id=(tgt,),
                device_id_type=pl.DeviceIdType.MESH,
            )
            rdma.start()
            inflight.append(rdma)
    for rdma in inflight:
        rdma.wait()
    for st in stores:
        st.wait()


def kernel(x, w_mat):
    out, _ = pl.pallas_call(
        _body,
        in_specs=[
            pl.BlockSpec(memory_space=pltpu.HBM),
            pl.BlockSpec(memory_space=pltpu.VMEM),
        ],
        out_specs=[
            pl.BlockSpec(memory_space=pltpu.HBM),
            pl.BlockSpec(memory_space=pltpu.HBM),
        ],
        out_shape=[
            jax.ShapeDtypeStruct((M, N), jnp.float32),
            jax.ShapeDtypeStruct((2, N_DEV - 1, CH, HALF), jnp.float32),
        ],
        scratch_shapes=[
            pltpu.VMEM((2, CH, HALF), jnp.float32),
            pltpu.VMEM((2, CH, HALF), jnp.float32),
            pltpu.VMEM((2, CH, K), jnp.float32),
            pltpu.SemaphoreType.DMA((2, N_DEV - 1)),
            pltpu.SemaphoreType.DMA((2, N_DEV - 1)),
            pltpu.SemaphoreType.DMA((2, N_DEV - 1)),
            pltpu.SemaphoreType.DMA((2, N_DEV - 1)),
            pltpu.SemaphoreType.DMA((8,)),
        ],
        compiler_params=pltpu.CompilerParams(
            collective_id=0, vmem_limit_bytes=100 * 1024 * 1024
        ),
    )(x, w_mat)
    return out


# device time: 1362164 ns/iter; 2.1222x vs baseline; 1.0614x over previous
import jax
import jax.numpy as jnp
from jax import lax
from jax.experimental import pallas as pl
from jax.experimental.pallas import tpu as pltpu

N_DEV = 8
M, K, N = 8192, 1024, 4096
CH = M // N_DEV
HALF = N // 2
SUB = CH // 2


def _gelu(v):
    c = 0.7978845608028654
    return 0.5 * v * (1.0 + jnp.tanh(c * (v + 0.044715 * v * v * v)))


def _body(x_ref, w_ref, o_ref, comm_ref,
          sv, vp, vx,
          rs_send, rs_recv, ag_send, ag_recv, cp_sem):
    me = lax.axis_index("i")
    right = lax.rem(me + 1, N_DEV)
    left = lax.rem(me + N_DEV - 1, N_DEV)

    barrier = pltpu.get_barrier_semaphore()
    for nbr in (left, right):
        pl.semaphore_signal(
            barrier, inc=1, device_id=(nbr,),
            device_id_type=pl.DeviceIdType.MESH,
        )

    def xchunk(idx):
        return x_ref.at[pl.ds(idx * CH, CH), :]

    def wcols(d):
        return w_ref[:, d * HALF:(d + 1) * HALF]

    def osub(idx, d, h):
        return o_ref.at[pl.ds(idx * CH + h * SUB, SUB),
                        pl.ds(d * HALF, HALF)]

    dirs = ((0, 1, right), (1, -1, left))
    rings = tuple((h,) + dd for h in (0, 1) for dd in dirs)

    def rs_rdma(d, h, tgt, s):
        return pltpu.make_async_remote_copy(
            src_ref=sv.at[d, pl.ds(h * SUB, SUB), :],
            dst_ref=comm_ref.at[d, s, pl.ds(h * SUB, SUB), :],
            send_sem=rs_send.at[d, s, h],
            recv_sem=rs_recv.at[d, s, h],
            device_id=(tgt,),
            device_id_type=pl.DeviceIdType.MESH,
        )

    c0 = pltpu.make_async_copy(xchunk(me), vx.at[0], cp_sem.at[4])
    c0.start()
    c0.wait()
    inflight = {}
    first = True
    for h, d, disp, tgt in rings:
        sv[d, h * SUB:(h + 1) * SUB, :] = jnp.dot(
            vx[0, h * SUB:(h + 1) * SUB, :], wcols(d),
            preferred_element_type=jnp.float32)
        if first:
            pl.semaphore_wait(barrier, 2)
            first = False
        rdma = rs_rdma(d, h, tgt, 0)
        rdma.start()
        inflight[(d, h)] = rdma

    def mm_partials(s):
        xc = []
        for d, disp, tgt in dirs:
            idx = lax.rem(me - disp * (s + 1) + 2 * N_DEV, N_DEV)
            c = pltpu.make_async_copy(xchunk(idx), vx.at[d], cp_sem.at[4 + d])
            c.start()
            xc.append(c)
        for d, disp, tgt in dirs:
            xc[d].wait()
            vp[d, :, :] = jnp.dot(
                vx[d], wcols(d), preferred_element_type=jnp.float32)

    mm_partials(0)
    for s in range(1, N_DEV - 1):
        nxt = {}
        for h, d, disp, tgt in rings:
            inflight[(d, h)].wait()
            c1 = pltpu.make_async_copy(
                comm_ref.at[d, s - 1, pl.ds(h * SUB, SUB), :],
                sv.at[d, pl.ds(h * SUB, SUB), :],
                cp_sem.at[2 * d + h])
            c1.start()
            c1.wait()
            rows = slice(h * SUB, (h + 1) * SUB)
            sv[d, rows, :] = sv[d, rows, :] + vp[d, rows, :]
            rdma = rs_rdma(d, h, tgt, s)
            rdma.start()
            nxt[(d, h)] = rdma
        inflight = nxt
        mm_partials(s)

    stores = []
    ag_inflight = {}
    for h, d, disp, tgt in rings:
        r = lax.rem(me + disp + N_DEV, N_DEV)
        inflight[(d, h)].wait()
        c1 = pltpu.make_async_copy(
            comm_ref.at[d, N_DEV - 2, pl.ds(h * SUB, SUB), :],
            sv.at[d, pl.ds(h * SUB, SUB), :],
            cp_sem.at[2 * d + h])
        c1.start()
        c1.wait()
        rows = slice(h * SUB, (h + 1) * SUB)
        sv[d, rows, :] = _gelu(sv[d, rows, :] + vp[d, rows, :])
        rdma = pltpu.make_async_remote_copy(
            src_ref=sv.at[d, pl.ds(h * SUB, SUB), :],
            dst_ref=osub(r, d, h),
            send_sem=ag_send.at[d, 0, h],
            recv_sem=ag_recv.at[d, 0, h],
            device_id=(tgt,),
            device_id_type=pl.DeviceIdType.MESH,
        )
        rdma.start()
        ag_inflight[(d, h)] = rdma
        st = pltpu.make_async_copy(
            sv.at[d, pl.ds(h * SUB, SUB), :], osub(r, d, h),
            cp_sem.at[8 + 2 * d + h])
        st.start()
        stores.append(st)

    for s in range(1, N_DEV - 1):
        nxt = {}
        for h, d, disp, tgt in rings:
            ag_inflight[(d, h)].wait()
            idx = lax.rem(me + disp - disp * s + 2 * N_DEV, N_DEV)
            rdma = pltpu.make_async_remote_copy(
                src_ref=osub(idx, d, h),
                dst_ref=osub(idx, d, h),
                send_sem=ag_send.at[d, s, h],
                recv_sem=ag_recv.at[d, s, h],
                device_id=(tgt,),
                device_id_type=pl.DeviceIdType.MESH,
            )
            rdma.start()
            nxt[(d, h)] = rdma
        ag_inflight = nxt
    for rdma in ag_inflight.values():
        rdma.wait()
    for st in stores:
        st.wait()


def kernel(x, w_mat):
    out, _ = pl.pallas_call(
        _body,
        in_specs=[
            pl.BlockSpec(memory_space=pltpu.HBM),
            pl.BlockSpec(memory_space=pltpu.VMEM),
        ],
        out_specs=[
            pl.BlockSpec(memory_space=pltpu.HBM),
            pl.BlockSpec(memory_space=pltpu.HBM),
        ],
        out_shape=[
            jax.ShapeDtypeStruct((M, N), jnp.float32),
            jax.ShapeDtypeStruct((2, N_DEV - 1, CH, HALF), jnp.float32),
        ],
        scratch_shapes=[
            pltpu.VMEM((2, CH, HALF), jnp.float32),
            pltpu.VMEM((2, CH, HALF), jnp.float32),
            pltpu.VMEM((2, CH, K), jnp.float32),
            pltpu.SemaphoreType.DMA((2, N_DEV - 1, 2)),
            pltpu.SemaphoreType.DMA((2, N_DEV - 1, 2)),
            pltpu.SemaphoreType.DMA((2, N_DEV - 1, 2)),
            pltpu.SemaphoreType.DMA((2, N_DEV - 1, 2)),
            pltpu.SemaphoreType.DMA((12,)),
        ],
        compiler_params=pltpu.CompilerParams(
            collective_id=0, vmem_limit_bytes=100 * 1024 * 1024
        ),
    )(x, w_mat)
    return out
